# baseline (speedup 1.0000x reference)
import functools

import jax
import jax.numpy as jnp
from jax import lax
from jax.experimental import pallas as pl
from jax.experimental.pallas import tpu as pltpu
from jax.experimental.pallas import tpu_sc as plsc

_NC, _NS, _L = 2, 16, 16
_NW = _NC * _NS
_CHUNK = 16


@functools.cache
def _build_sc_kernel(B, S, D, dtype):
    rows_per_w = S // _NW
    n_chunks = rows_per_w // _CHUNK
    vecs_per_row = D // _L

    mesh = plsc.VectorSubcoreMesh(core_axis_name="c", subcore_axis_name="s")

    @functools.partial(
        pl.kernel,
        out_type=jax.ShapeDtypeStruct((B, S, D), dtype),
        mesh=mesh,
        scratch_types=[
            pltpu.VMEM((_CHUNK, D), jnp.float32),
            pltpu.VMEM((B, _CHUNK, D), jnp.float32),
        ],
    )
    def sc_kernel(in_hbm, emb_hbm, out_hbm, emb_v, io_v):
        wid = lax.axis_index("s") * _NC + lax.axis_index("c")
        base = wid * rows_per_w

        @pl.loop(0, n_chunks)
        def _chunk(ci):
            row0 = base + ci * _CHUNK
            pltpu.sync_copy(emb_hbm.at[pl.ds(row0, _CHUNK)], emb_v)
            for b in range(B):
                pltpu.sync_copy(in_hbm.at[b, pl.ds(row0, _CHUNK)], io_v.at[b])

            @pl.loop(0, _CHUNK)
            def _row(r):
                @pl.loop(0, vecs_per_row, unroll=8)
                def _col(cv):
                    sl = pl.ds(cv * _L, _L)
                    e = emb_v[r, sl]
                    for b in range(B):
                        plsc.addupdate(io_v.at[b, r, sl], e)

            for b in range(B):
                pltpu.sync_copy(io_v.at[b], out_hbm.at[b, pl.ds(row0, _CHUNK)])

    return sc_kernel


def kernel(inputs, embeddings):
    B, S, D = inputs.shape
    pos = embeddings[:S]
    return _build_sc_kernel(B, S, D, inputs.dtype)(inputs, pos)
